# baseline (speedup 1.0000x reference)
import jax
import jax.numpy as jnp
from jax import lax
from jax.experimental import pallas as pl

_N = 35
_E = 1190
_EP = 1280
_V = 6
_VA = 7
_IB = 16


def _dot(a, b):
    return lax.dot_general(a, b, (((1,), (0,)), ((), ())),
                           preferred_element_type=jnp.float32)


def _nnconv128(ea, xj, W_ref, kd, fold):
    bw = _IB * 128
    xjh = xj.astype(jnp.bfloat16)

    def body(i0, acc):
        wv = W_ref[:, pl.ds(i0 * bw, bw)]
        genb = jnp.maximum(_dot(ea, wv), 0.0)
        sel = (kd == i0 * _IB).astype(jnp.bfloat16)
        xb = _dot(xjh, sel)
        p = (xb * genb).astype(jnp.bfloat16)
        return acc + _dot(p, fold)

    return lax.fori_loop(0, 128 // _IB, body,
                         jnp.zeros((_EP, 128), jnp.float32))


def _nnconv64(ea, xj, W_ref, kd, fold):
    bw = _IB * 64
    xjh = xj.astype(jnp.bfloat16)

    def body(i0, acc):
        wv = W_ref[:, pl.ds(i0 * bw, bw)]
        genb = jnp.maximum(_dot(ea, wv), 0.0)
        sel = (kd == i0 * _IB).astype(jnp.bfloat16)
        xb = _dot(xjh, sel)
        p = (xb * genb).astype(jnp.bfloat16)
        return acc + _dot(p, fold)

    return lax.fori_loop(0, 128 // _IB, body,
                         jnp.zeros((_EP, 64), jnp.float32))


def _dgn_kernel(ea_ref, src_ref, dst_ref, x_ref,
                W1_ref, root1_ref, bias1_ref,
                W2_ref, root2_ref, bias2_ref,
                W3_ref, root3_ref, bias3_ref,
                out_ref):
    f32 = jnp.float32
    ea = ea_ref[:]
    src = src_ref[:]
    dst = dst_ref[:]

    col = lax.broadcasted_iota(jnp.int32, (_EP, _N), 1)
    G = (src == col).astype(f32)
    row = lax.broadcasted_iota(jnp.int32, (_N, _EP), 0)
    S = (row == dst).astype(f32)
    cnt = jnp.sum(S, axis=1, keepdims=True)
    inv = 1.0 / jnp.maximum(cnt, 1.0)

    def kdmat(cout):
        bw = _IB * cout
        k_i = lax.broadcasted_iota(jnp.int32, (128, bw), 0)
        c_i = lax.broadcasted_iota(jnp.int32, (128, bw), 1)
        return k_i - c_i // cout

    x0 = x_ref[:]
    gen = jnp.maximum(_dot(ea, W1_ref[:]), 0.0)
    xj = _dot(G, x0)
    msg = xj * gen
    agg = _dot(S, msg) * inv
    h = jnp.maximum(_dot(x0, root1_ref[:]) + agg + bias1_ref[:], 0.0)

    xj = _dot(G, h)
    c2 = lax.broadcasted_iota(jnp.int32, (_IB * 128, 128), 0)
    o2 = lax.broadcasted_iota(jnp.int32, (_IB * 128, 128), 1)
    fold2 = (c2 % 128 == o2).astype(jnp.bfloat16)
    msg = _nnconv128(ea, xj, W2_ref, kdmat(128), fold2)
    agg = _dot(S, msg) * inv
    h = jnp.maximum(_dot(h, root2_ref[:]) + agg + bias2_ref[:], 0.0)

    xj = _dot(G, h)
    c3 = lax.broadcasted_iota(jnp.int32, (_IB * 64, 64), 0)
    o3 = lax.broadcasted_iota(jnp.int32, (_IB * 64, 64), 1)
    fold = (c3 % 64 == o3).astype(jnp.bfloat16)
    msg = _nnconv64(ea, xj, W3_ref, kdmat(64), fold)
    agg = _dot(S, msg) * inv
    h = jnp.maximum(_dot(h, root3_ref[:]) + agg + bias3_ref[:], 0.0)

    d = jnp.abs(h[:, None, :] - h[None, :, :])
    out_ref[:] = jnp.sum(d, axis=2)


@jax.jit
def kernel(x, edge_attr, edge_index, W1, b1, root1, bias1,
           W2, b2, root2, bias2, W3, b3, root3, bias3):
    f32 = jnp.float32
    bf16 = jnp.bfloat16
    ea = (jnp.zeros((_EP, _VA), f32)
          .at[:_E, :_V].set(edge_attr)
          .at[:, _V].set(1.0)).astype(bf16)
    src = jnp.full((_EP, 1), _N, jnp.int32).at[:_E, 0].set(edge_index[0])
    dst = jnp.full((1, _EP), _N, jnp.int32).at[0, :_E].set(edge_index[1])

    W1a = jnp.concatenate([W1, b1.reshape(1, -1)], axis=0).astype(bf16)
    W2a = jnp.concatenate([W2, b2.reshape(1, -1)], axis=0).astype(bf16)
    W3a = jnp.concatenate([W3, b3.reshape(1, -1)], axis=0).astype(bf16)

    out = pl.pallas_call(
        _dgn_kernel,
        out_shape=jax.ShapeDtypeStruct((_N, _N), f32),
    )(ea, src, dst, x,
      W1a, root1, bias1.reshape(1, -1),
      W2a, root2, bias2.reshape(1, -1),
      W3a, root3, bias3.reshape(1, -1))
    return out


# speedup vs baseline: 1.1822x; 1.1822x over previous
import jax
import jax.numpy as jnp
from jax import lax
from jax.experimental import pallas as pl
from jax.experimental.pallas import tpu as pltpu

_N = 35
_E = 1190
_EP = 1280
_V = 6
_VA = 7
_IB = 16


def _dot(a, b):
    return lax.dot_general(a, b, (((1,), (0,)), ((), ())),
                           preferred_element_type=jnp.float32)


def _nnconv128(ea, xj, W_ref):
    bw = _IB * 128

    def body(i0, acc):
        wv = W_ref[:, pl.ds(i0 * bw, bw)]
        genb = jnp.maximum(_dot(ea, wv), 0.0)
        xjr = pltpu.roll(xj, -i0 * _IB, axis=1)
        for di in range(_IB):
            acc = acc + xjr[:, di:di + 1] * genb[:, di * 128:(di + 1) * 128]
        return acc

    return lax.fori_loop(0, 128 // _IB, body,
                         jnp.zeros((_EP, 128), jnp.float32))


def _nnconv64(ea, xj, W_ref, kd, fold):
    bw = _IB * 64
    xjh = xj.astype(jnp.bfloat16)

    def body(i0, acc):
        wv = W_ref[:, pl.ds(i0 * bw, bw)]
        genb = jnp.maximum(_dot(ea, wv), 0.0)
        sel = (kd == i0 * _IB).astype(jnp.bfloat16)
        xb = _dot(xjh, sel)
        p = (xb * genb).astype(jnp.bfloat16)
        return acc + _dot(p, fold)

    return lax.fori_loop(0, 128 // _IB, body,
                         jnp.zeros((_EP, 64), jnp.float32))


def _dgn_kernel(ea_ref, src_ref, dst_ref, x_ref,
                W1_ref, root1_ref, bias1_ref,
                W2_ref, root2_ref, bias2_ref,
                W3_ref, root3_ref, bias3_ref,
                out_ref):
    f32 = jnp.float32
    ea = ea_ref[:]
    src = src_ref[:]
    dst = dst_ref[:]

    col = lax.broadcasted_iota(jnp.int32, (_EP, _N), 1)
    G = (src == col).astype(f32)
    row = lax.broadcasted_iota(jnp.int32, (_N, _EP), 0)
    S = (row == dst).astype(f32)
    cnt = jnp.sum(S, axis=1, keepdims=True)
    inv = 1.0 / jnp.maximum(cnt, 1.0)

    def kdmat(cout):
        bw = _IB * cout
        k_i = lax.broadcasted_iota(jnp.int32, (128, bw), 0)
        c_i = lax.broadcasted_iota(jnp.int32, (128, bw), 1)
        return k_i - c_i // cout

    x0 = x_ref[:]
    gen = jnp.maximum(_dot(ea, W1_ref[:]), 0.0)
    xj = _dot(G, x0)
    msg = xj * gen
    agg = _dot(S, msg) * inv
    h = jnp.maximum(_dot(x0, root1_ref[:]) + agg + bias1_ref[:], 0.0)

    xj = _dot(G, h)
    msg = _nnconv128(ea, xj, W2_ref)
    agg = _dot(S, msg) * inv
    h = jnp.maximum(_dot(h, root2_ref[:]) + agg + bias2_ref[:], 0.0)

    xj = _dot(G, h)
    c3 = lax.broadcasted_iota(jnp.int32, (_IB * 64, 64), 0)
    o3 = lax.broadcasted_iota(jnp.int32, (_IB * 64, 64), 1)
    fold = (c3 % 64 == o3).astype(jnp.bfloat16)
    msg = _nnconv64(ea, xj, W3_ref, kdmat(64), fold)
    agg = _dot(S, msg) * inv
    h = jnp.maximum(_dot(h, root3_ref[:]) + agg + bias3_ref[:], 0.0)

    d = jnp.abs(h[:, None, :] - h[None, :, :])
    out_ref[:] = jnp.sum(d, axis=2)


@jax.jit
def kernel(x, edge_attr, edge_index, W1, b1, root1, bias1,
           W2, b2, root2, bias2, W3, b3, root3, bias3):
    f32 = jnp.float32
    bf16 = jnp.bfloat16
    ea = (jnp.zeros((_EP, _VA), f32)
          .at[:_E, :_V].set(edge_attr)
          .at[:, _V].set(1.0)).astype(bf16)
    src = jnp.full((_EP, 1), _N, jnp.int32).at[:_E, 0].set(edge_index[0])
    dst = jnp.full((1, _EP), _N, jnp.int32).at[0, :_E].set(edge_index[1])

    W1a = jnp.concatenate([W1, b1.reshape(1, -1)], axis=0).astype(bf16)
    W2a = jnp.concatenate([W2, b2.reshape(1, -1)], axis=0).astype(bf16)
    W3a = jnp.concatenate([W3, b3.reshape(1, -1)], axis=0).astype(bf16)

    out = pl.pallas_call(
        _dgn_kernel,
        out_shape=jax.ShapeDtypeStruct((_N, _N), f32),
    )(ea, src, dst, x,
      W1a, root1, bias1.reshape(1, -1),
      W2a, root2, bias2.reshape(1, -1),
      W3a, root3, bias3.reshape(1, -1))
    return out


# speedup vs baseline: 1.1960x; 1.0116x over previous
import jax
import jax.numpy as jnp
from jax import lax
from jax.experimental import pallas as pl

_N = 35
_E = 1190
_EP = 1280
_V = 6
_VA = 7
_IB = 16


def _dot(a, b):
    return lax.dot_general(a, b, (((1,), (0,)), ((), ())),
                           preferred_element_type=jnp.float32)


def _nnconv128(ea, xj, W_ref, kd):
    bw = _IB * 128
    xjh = xj.astype(jnp.bfloat16)

    def body(i0, acc):
        wv = W_ref[:, pl.ds(i0 * bw, bw)]
        genb = jnp.maximum(_dot(ea, wv), 0.0)
        sel = (kd == i0 * _IB).astype(jnp.bfloat16)
        xb = _dot(xjh, sel)
        p = xb * genb
        for di in range(_IB):
            acc = acc + p[:, di * 128:(di + 1) * 128]
        return acc

    return lax.fori_loop(0, 128 // _IB, body,
                         jnp.zeros((_EP, 128), jnp.float32))


def _nnconv64(ea, xj, W_ref, kd, fold):
    bw = _IB * 64
    xjh = xj.astype(jnp.bfloat16)

    def body(i0, acc):
        wv = W_ref[:, pl.ds(i0 * bw, bw)]
        genb = jnp.maximum(_dot(ea, wv), 0.0)
        sel = (kd == i0 * _IB).astype(jnp.bfloat16)
        xb = _dot(xjh, sel)
        p = (xb * genb).astype(jnp.bfloat16)
        return acc + _dot(p, fold)

    return lax.fori_loop(0, 128 // _IB, body,
                         jnp.zeros((_EP, 64), jnp.float32))


def _dgn_kernel(ea_ref, src_ref, dst_ref, x_ref,
                W1_ref, root1_ref, bias1_ref,
                W2_ref, root2_ref, bias2_ref,
                W3_ref, root3_ref, bias3_ref,
                out_ref):
    f32 = jnp.float32
    ea = ea_ref[:]
    src = src_ref[:]
    dst = dst_ref[:]

    col = lax.broadcasted_iota(jnp.int32, (_EP, _N), 1)
    G = (src == col).astype(f32)
    row = lax.broadcasted_iota(jnp.int32, (_N, _EP), 0)
    S = (row == dst).astype(f32)
    cnt = jnp.sum(S, axis=1, keepdims=True)
    inv = 1.0 / jnp.maximum(cnt, 1.0)

    def kdmat(cout):
        bw = _IB * cout
        k_i = lax.broadcasted_iota(jnp.int32, (128, bw), 0)
        c_i = lax.broadcasted_iota(jnp.int32, (128, bw), 1)
        return k_i - c_i // cout

    x0 = x_ref[:]
    gen = jnp.maximum(_dot(ea, W1_ref[:]), 0.0)
    xj = _dot(G, x0)
    msg = xj * gen
    agg = _dot(S, msg) * inv
    h = jnp.maximum(_dot(x0, root1_ref[:]) + agg + bias1_ref[:], 0.0)

    xj = _dot(G, h)
    msg = _nnconv128(ea, xj, W2_ref, kdmat(128))
    agg = _dot(S, msg) * inv
    h = jnp.maximum(_dot(h, root2_ref[:]) + agg + bias2_ref[:], 0.0)

    xj = _dot(G, h)
    c3 = lax.broadcasted_iota(jnp.int32, (_IB * 64, 64), 0)
    o3 = lax.broadcasted_iota(jnp.int32, (_IB * 64, 64), 1)
    fold = (c3 % 64 == o3).astype(jnp.bfloat16)
    msg = _nnconv64(ea, xj, W3_ref, kdmat(64), fold)
    agg = _dot(S, msg) * inv
    h = jnp.maximum(_dot(h, root3_ref[:]) + agg + bias3_ref[:], 0.0)

    d = jnp.abs(h[:, None, :] - h[None, :, :])
    out_ref[:] = jnp.sum(d, axis=2)


@jax.jit
def kernel(x, edge_attr, edge_index, W1, b1, root1, bias1,
           W2, b2, root2, bias2, W3, b3, root3, bias3):
    f32 = jnp.float32
    bf16 = jnp.bfloat16
    ea = (jnp.zeros((_EP, _VA), f32)
          .at[:_E, :_V].set(edge_attr)
          .at[:, _V].set(1.0)).astype(bf16)
    src = jnp.full((_EP, 1), _N, jnp.int32).at[:_E, 0].set(edge_index[0])
    dst = jnp.full((1, _EP), _N, jnp.int32).at[0, :_E].set(edge_index[1])

    W1a = jnp.concatenate([W1, b1.reshape(1, -1)], axis=0).astype(bf16)
    W2a = jnp.concatenate([W2, b2.reshape(1, -1)], axis=0).astype(bf16)
    W3a = jnp.concatenate([W3, b3.reshape(1, -1)], axis=0).astype(bf16)

    out = pl.pallas_call(
        _dgn_kernel,
        out_shape=jax.ShapeDtypeStruct((_N, _N), f32),
    )(ea, src, dst, x,
      W1a, root1, bias1.reshape(1, -1),
      W2a, root2, bias2.reshape(1, -1),
      W3a, root3, bias3.reshape(1, -1))
    return out


# speedup vs baseline: 1.2116x; 1.0131x over previous
import jax
import jax.numpy as jnp
from jax import lax
from jax.experimental import pallas as pl

_N = 35
_E = 1190
_EP = 1280
_V = 6
_VA = 7
_IB = 16
_IB3 = 32


def _dot(a, b):
    return lax.dot_general(a, b, (((1,), (0,)), ((), ())),
                           preferred_element_type=jnp.float32)


def _nnconv128(ea, xj, W_ref, kd):
    bw = _IB * 128
    xjh = xj.astype(jnp.bfloat16)

    def body(i0, acc):
        wv = W_ref[:, pl.ds(i0 * bw, bw)]
        genb = jnp.maximum(_dot(ea, wv), 0.0)
        sel = (kd == i0 * _IB).astype(jnp.bfloat16)
        xb = _dot(xjh, sel)
        p = xb * genb
        for di in range(_IB):
            acc = acc + p[:, di * 128:(di + 1) * 128]
        return acc

    return lax.fori_loop(0, 128 // _IB, body,
                         jnp.zeros((_EP, 128), jnp.float32))


def _nnconv64(ea, xj, W_ref, kd, fold):
    bw = _IB3 * 64
    xjh = xj.astype(jnp.bfloat16)

    def body(i0, acc):
        wv = W_ref[:, pl.ds(i0 * bw, bw)]
        genb = jnp.maximum(_dot(ea, wv), 0.0)
        sel = (kd == i0 * _IB3).astype(jnp.bfloat16)
        xb = _dot(xjh, sel)
        p = (xb * genb).astype(jnp.bfloat16)
        return acc + _dot(p, fold)

    return lax.fori_loop(0, 128 // _IB3, body,
                         jnp.zeros((_EP, 64), jnp.float32))


def _dgn_kernel(ea_ref, src_ref, dst_ref, x_ref,
                W1_ref, root1_ref, bias1_ref,
                W2_ref, root2_ref, bias2_ref,
                W3_ref, root3_ref, bias3_ref,
                out_ref):
    f32 = jnp.float32
    ea = ea_ref[:]
    src = src_ref[:]
    dst = dst_ref[:]

    col = lax.broadcasted_iota(jnp.int32, (_EP, _N), 1)
    G = (src == col).astype(f32)
    row = lax.broadcasted_iota(jnp.int32, (_N, _EP), 0)
    S = (row == dst).astype(f32)
    cnt = jnp.sum(S, axis=1, keepdims=True)
    inv = 1.0 / jnp.maximum(cnt, 1.0)

    def kdmat(cout, ib):
        bw = ib * cout
        k_i = lax.broadcasted_iota(jnp.int32, (128, bw), 0)
        c_i = lax.broadcasted_iota(jnp.int32, (128, bw), 1)
        return k_i - c_i // cout

    x0 = x_ref[:]
    gen = jnp.maximum(_dot(ea, W1_ref[:]), 0.0)
    xj = _dot(G, x0)
    msg = xj * gen
    agg = _dot(S, msg) * inv
    h = jnp.maximum(_dot(x0, root1_ref[:]) + agg + bias1_ref[:], 0.0)

    xj = _dot(G, h)
    msg = _nnconv128(ea, xj, W2_ref, kdmat(128, _IB))
    agg = _dot(S, msg) * inv
    h = jnp.maximum(_dot(h, root2_ref[:]) + agg + bias2_ref[:], 0.0)

    xj = _dot(G, h)
    c3 = lax.broadcasted_iota(jnp.int32, (_IB3 * 64, 64), 0)
    o3 = lax.broadcasted_iota(jnp.int32, (_IB3 * 64, 64), 1)
    fold = (c3 % 64 == o3).astype(jnp.bfloat16)
    msg = _nnconv64(ea, xj, W3_ref, kdmat(64, _IB3), fold)
    agg = _dot(S, msg) * inv
    h = jnp.maximum(_dot(h, root3_ref[:]) + agg + bias3_ref[:], 0.0)

    d = jnp.abs(h[:, None, :] - h[None, :, :])
    out_ref[:] = jnp.sum(d, axis=2)


@jax.jit
def kernel(x, edge_attr, edge_index, W1, b1, root1, bias1,
           W2, b2, root2, bias2, W3, b3, root3, bias3):
    f32 = jnp.float32
    bf16 = jnp.bfloat16
    ea = (jnp.zeros((_EP, _VA), f32)
          .at[:_E, :_V].set(edge_attr)
          .at[:, _V].set(1.0)).astype(bf16)
    src = jnp.full((_EP, 1), _N, jnp.int32).at[:_E, 0].set(edge_index[0])
    dst = jnp.full((1, _EP), _N, jnp.int32).at[0, :_E].set(edge_index[1])

    W1a = jnp.concatenate([W1, b1.reshape(1, -1)], axis=0).astype(bf16)
    W2a = jnp.concatenate([W2, b2.reshape(1, -1)], axis=0).astype(bf16)
    W3a = jnp.concatenate([W3, b3.reshape(1, -1)], axis=0).astype(bf16)

    out = pl.pallas_call(
        _dgn_kernel,
        out_shape=jax.ShapeDtypeStruct((_N, _N), f32),
    )(ea, src, dst, x,
      W1a, root1, bias1.reshape(1, -1),
      W2a, root2, bias2.reshape(1, -1),
      W3a, root3, bias3.reshape(1, -1))
    return out


# speedup vs baseline: 1.2283x; 1.0138x over previous
import jax
import jax.numpy as jnp
from jax import lax
from jax.experimental import pallas as pl

_N = 35
_E = 1190
_EP = 1280
_V = 6
_VA = 7
_IB = 32
_IB3 = 32


def _dot(a, b):
    return lax.dot_general(a, b, (((1,), (0,)), ((), ())),
                           preferred_element_type=jnp.float32)


def _nnconv128(ea, xj, W_ref, kd):
    bw = _IB * 128
    xjh = xj.astype(jnp.bfloat16)

    def body(i0, acc):
        wv = W_ref[:, pl.ds(i0 * bw, bw)]
        genb = jnp.maximum(_dot(ea, wv), 0.0)
        sel = (kd == i0 * _IB).astype(jnp.bfloat16)
        xb = _dot(xjh, sel)
        p = xb * genb
        for di in range(_IB):
            acc = acc + p[:, di * 128:(di + 1) * 128]
        return acc

    return lax.fori_loop(0, 128 // _IB, body,
                         jnp.zeros((_EP, 128), jnp.float32))


def _nnconv64(ea, xj, W_ref, kd, fold):
    bw = _IB3 * 64
    xjh = xj.astype(jnp.bfloat16)

    def body(i0, acc):
        wv = W_ref[:, pl.ds(i0 * bw, bw)]
        genb = jnp.maximum(_dot(ea, wv), 0.0)
        sel = (kd == i0 * _IB3).astype(jnp.bfloat16)
        xb = _dot(xjh, sel)
        p = (xb * genb).astype(jnp.bfloat16)
        return acc + _dot(p, fold)

    return lax.fori_loop(0, 128 // _IB3, body,
                         jnp.zeros((_EP, 64), jnp.float32))


def _dgn_kernel(ea_ref, src_ref, dst_ref, x_ref,
                W1_ref, root1_ref, bias1_ref,
                W2_ref, root2_ref, bias2_ref,
                W3_ref, root3_ref, bias3_ref,
                out_ref):
    f32 = jnp.float32
    ea = ea_ref[:]
    src = src_ref[:]
    dst = dst_ref[:]

    col = lax.broadcasted_iota(jnp.int32, (_EP, _N), 1)
    G = (src == col).astype(f32)
    row = lax.broadcasted_iota(jnp.int32, (_N, _EP), 0)
    S = (row == dst).astype(f32)
    cnt = jnp.sum(S, axis=1, keepdims=True)
    inv = 1.0 / jnp.maximum(cnt, 1.0)

    def kdmat(cout, ib):
        bw = ib * cout
        k_i = lax.broadcasted_iota(jnp.int32, (128, bw), 0)
        c_i = lax.broadcasted_iota(jnp.int32, (128, bw), 1)
        return k_i - c_i // cout

    x0 = x_ref[:]
    gen = jnp.maximum(_dot(ea, W1_ref[:]), 0.0)
    xj = _dot(G, x0)
    msg = xj * gen
    agg = _dot(S, msg) * inv
    h = jnp.maximum(_dot(x0, root1_ref[:]) + agg + bias1_ref[:], 0.0)

    xj = _dot(G, h)
    msg = _nnconv128(ea, xj, W2_ref, kdmat(128, _IB))
    agg = _dot(S, msg) * inv
    h = jnp.maximum(_dot(h, root2_ref[:]) + agg + bias2_ref[:], 0.0)

    xj = _dot(G, h)
    c3 = lax.broadcasted_iota(jnp.int32, (_IB3 * 64, 64), 0)
    o3 = lax.broadcasted_iota(jnp.int32, (_IB3 * 64, 64), 1)
    fold = (c3 % 64 == o3).astype(jnp.bfloat16)
    msg = _nnconv64(ea, xj, W3_ref, kdmat(64, _IB3), fold)
    agg = _dot(S, msg) * inv
    h = jnp.maximum(_dot(h, root3_ref[:]) + agg + bias3_ref[:], 0.0)

    d = jnp.abs(h[:, None, :] - h[None, :, :])
    out_ref[:] = jnp.sum(d, axis=2)


@jax.jit
def kernel(x, edge_attr, edge_index, W1, b1, root1, bias1,
           W2, b2, root2, bias2, W3, b3, root3, bias3):
    f32 = jnp.float32
    bf16 = jnp.bfloat16
    ea = (jnp.zeros((_EP, _VA), f32)
          .at[:_E, :_V].set(edge_attr)
          .at[:, _V].set(1.0)).astype(bf16)
    src = jnp.full((_EP, 1), _N, jnp.int32).at[:_E, 0].set(edge_index[0])
    dst = jnp.full((1, _EP), _N, jnp.int32).at[0, :_E].set(edge_index[1])

    W1a = jnp.concatenate([W1, b1.reshape(1, -1)], axis=0).astype(bf16)
    W2a = jnp.concatenate([W2, b2.reshape(1, -1)], axis=0).astype(bf16)
    W3a = jnp.concatenate([W3, b3.reshape(1, -1)], axis=0).astype(bf16)

    out = pl.pallas_call(
        _dgn_kernel,
        out_shape=jax.ShapeDtypeStruct((_N, _N), f32),
    )(ea, src, dst, x,
      W1a, root1, bias1.reshape(1, -1),
      W2a, root2, bias2.reshape(1, -1),
      W3a, root3, bias3.reshape(1, -1))
    return out


# speedup vs baseline: 1.2353x; 1.0057x over previous
import jax
import jax.numpy as jnp
from jax import lax
from jax.experimental import pallas as pl

_N = 35
_E = 1190
_EP = 1280
_V = 6
_VA = 7
_IB = 32
_IB3 = 64


def _dot(a, b):
    return lax.dot_general(a, b, (((1,), (0,)), ((), ())),
                           preferred_element_type=jnp.float32)


def _nnconv128(ea, xj, W_ref, kd):
    bw = _IB * 128
    xjh = xj.astype(jnp.bfloat16)

    def body(i0, acc):
        wv = W_ref[:, pl.ds(i0 * bw, bw)]
        genb = jnp.maximum(_dot(ea, wv), 0.0)
        sel = (kd == i0 * _IB).astype(jnp.bfloat16)
        xb = _dot(xjh, sel)
        p = xb * genb
        for di in range(_IB):
            acc = acc + p[:, di * 128:(di + 1) * 128]
        return acc

    return lax.fori_loop(0, 128 // _IB, body,
                         jnp.zeros((_EP, 128), jnp.float32))


def _nnconv64(ea, xj, W_ref, kd, fold):
    bw = _IB3 * 64
    xjh = xj.astype(jnp.bfloat16)

    def body(i0, acc):
        wv = W_ref[:, pl.ds(i0 * bw, bw)]
        genb = jnp.maximum(_dot(ea, wv), 0.0)
        sel = (kd == i0 * _IB3).astype(jnp.bfloat16)
        xb = _dot(xjh, sel)
        p = (xb * genb).astype(jnp.bfloat16)
        return acc + _dot(p, fold)

    return lax.fori_loop(0, 128 // _IB3, body,
                         jnp.zeros((_EP, 64), jnp.float32))


def _dgn_kernel(ea_ref, src_ref, dst_ref, x_ref,
                W1_ref, root1_ref, bias1_ref,
                W2_ref, root2_ref, bias2_ref,
                W3_ref, root3_ref, bias3_ref,
                out_ref):
    f32 = jnp.float32
    ea = ea_ref[:]
    src = src_ref[:]
    dst = dst_ref[:]

    col = lax.broadcasted_iota(jnp.int32, (_EP, _N), 1)
    G = (src == col).astype(f32)
    row = lax.broadcasted_iota(jnp.int32, (_N, _EP), 0)
    S = (row == dst).astype(f32)
    cnt = jnp.sum(S, axis=1, keepdims=True)
    inv = 1.0 / jnp.maximum(cnt, 1.0)

    def kdmat(cout, ib):
        bw = ib * cout
        k_i = lax.broadcasted_iota(jnp.int32, (128, bw), 0)
        c_i = lax.broadcasted_iota(jnp.int32, (128, bw), 1)
        return k_i - c_i // cout

    x0 = x_ref[:]
    gen = jnp.maximum(_dot(ea, W1_ref[:]), 0.0)
    xj = _dot(G, x0)
    msg = xj * gen
    agg = _dot(S, msg) * inv
    h = jnp.maximum(_dot(x0, root1_ref[:]) + agg + bias1_ref[:], 0.0)

    xj = _dot(G, h)
    msg = _nnconv128(ea, xj, W2_ref, kdmat(128, _IB))
    agg = _dot(S, msg) * inv
    h = jnp.maximum(_dot(h, root2_ref[:]) + agg + bias2_ref[:], 0.0)

    xj = _dot(G, h)
    c3 = lax.broadcasted_iota(jnp.int32, (_IB3 * 64, 64), 0)
    o3 = lax.broadcasted_iota(jnp.int32, (_IB3 * 64, 64), 1)
    fold = (c3 % 64 == o3).astype(jnp.bfloat16)
    msg = _nnconv64(ea, xj, W3_ref, kdmat(64, _IB3), fold)
    agg = _dot(S, msg) * inv
    h = jnp.maximum(_dot(h, root3_ref[:]) + agg + bias3_ref[:], 0.0)

    d = jnp.abs(h[:, None, :] - h[None, :, :])
    out_ref[:] = jnp.sum(d, axis=2)


@jax.jit
def kernel(x, edge_attr, edge_index, W1, b1, root1, bias1,
           W2, b2, root2, bias2, W3, b3, root3, bias3):
    f32 = jnp.float32
    bf16 = jnp.bfloat16
    ea = (jnp.zeros((_EP, _VA), f32)
          .at[:_E, :_V].set(edge_attr)
          .at[:, _V].set(1.0)).astype(bf16)
    src = jnp.full((_EP, 1), _N, jnp.int32).at[:_E, 0].set(edge_index[0])
    dst = jnp.full((1, _EP), _N, jnp.int32).at[0, :_E].set(edge_index[1])

    W1a = jnp.concatenate([W1, b1.reshape(1, -1)], axis=0).astype(bf16)
    W2a = jnp.concatenate([W2, b2.reshape(1, -1)], axis=0).astype(bf16)
    W3a = jnp.concatenate([W3, b3.reshape(1, -1)], axis=0).astype(bf16)

    out = pl.pallas_call(
        _dgn_kernel,
        out_shape=jax.ShapeDtypeStruct((_N, _N), f32),
    )(ea, src, dst, x,
      W1a, root1, bias1.reshape(1, -1),
      W2a, root2, bias2.reshape(1, -1),
      W3a, root3, bias3.reshape(1, -1))
    return out
